# Initial kernel scaffold; baseline (speedup 1.0000x reference)
import jax
import jax.numpy as jnp
from jax.experimental import pallas as pl


def kernel(inputs, embeddings):
    raise NotImplementedError("write your pallas kernel here")


# speedup vs baseline: 2.9675x; 2.9675x over previous
import functools

import jax
import jax.numpy as jnp
from jax import lax
from jax.experimental import pallas as pl
from jax.experimental.pallas import tpu as pltpu
from jax.experimental.pallas import tpu_sc as plsc

VOCAB = 100000
D = 128
NUM_CORES = 2
NUM_SUBCORES = 16
NW = NUM_CORES * NUM_SUBCORES
CHUNK = 128


@functools.partial(jax.jit, static_argnums=(2,))
def _lookup(flat_idx, embeddings, n):
    b_per_w = n // NW
    n_chunks = b_per_w // CHUNK
    mesh = plsc.VectorSubcoreMesh(core_axis_name="c", subcore_axis_name="s")

    @functools.partial(
        pl.kernel,
        mesh=mesh,
        out_type=jax.ShapeDtypeStruct((n, D), jnp.float32),
        scratch_types=[
            pltpu.VMEM((b_per_w,), jnp.int32),
            pltpu.VMEM((CHUNK, D), jnp.float32),
            pltpu.SemaphoreType.DMA,
        ],
    )
    def k(idx_hbm, table_hbm, out_hbm, idx_v, rows_v, sem):
        wid = lax.axis_index("s") * NUM_CORES + lax.axis_index("c")
        base = wid * b_per_w
        pltpu.sync_copy(idx_hbm.at[pl.ds(base, b_per_w)], idx_v)

        def body(c, carry):
            pltpu.async_copy(
                table_hbm.at[idx_v.at[pl.ds(c * CHUNK, CHUNK)]], rows_v, sem
            ).wait()
            pltpu.sync_copy(rows_v, out_hbm.at[pl.ds(base + c * CHUNK, CHUNK)])
            return carry

        lax.fori_loop(0, n_chunks, body, 0)

    return k(flat_idx, embeddings)


def kernel(inputs, embeddings):
    shape = inputs.shape
    flat = jnp.reshape(inputs, (-1,)).astype(jnp.int32)
    out = _lookup(flat, embeddings, flat.shape[0])
    return jnp.reshape(out, tuple(shape) + (D,))


# speedup vs baseline: 3.1232x; 1.0525x over previous
import functools

import jax
import jax.numpy as jnp
from jax import lax
from jax.experimental import pallas as pl
from jax.experimental.pallas import tpu as pltpu
from jax.experimental.pallas import tpu_sc as plsc

VOCAB = 100000
D = 128
NUM_CORES = 2
NUM_SUBCORES = 16
NW = NUM_CORES * NUM_SUBCORES
CHUNK = 128


@functools.partial(jax.jit, static_argnums=(2,))
def _lookup(flat_idx, embeddings, n):
    b_per_w = n // NW
    n_chunks = b_per_w // CHUNK
    mesh = plsc.VectorSubcoreMesh(core_axis_name="c", subcore_axis_name="s")

    assert n_chunks % 2 == 0

    @functools.partial(
        pl.kernel,
        mesh=mesh,
        out_type=jax.ShapeDtypeStruct((n, D), jnp.float32),
        scratch_types=[
            pltpu.VMEM((b_per_w,), jnp.int32),
            pltpu.VMEM((CHUNK, D), jnp.float32),
            pltpu.VMEM((CHUNK, D), jnp.float32),
            pltpu.SemaphoreType.DMA,
            pltpu.SemaphoreType.DMA,
        ],
    )
    def k(idx_hbm, table_hbm, out_hbm, idx_v, rows0, rows1, sem0, sem1):
        wid = lax.axis_index("s") * NUM_CORES + lax.axis_index("c")
        base = wid * b_per_w
        pltpu.sync_copy(idx_hbm.at[pl.ds(base, b_per_w)], idx_v)

        bufs = (rows0, rows1)
        sems = (sem0, sem1)

        def gather(c, buf, sem):
            return pltpu.make_async_copy(
                table_hbm.at[idx_v.at[pl.ds(c * CHUNK, CHUNK)]], buf, sem
            )

        gather(0, rows0, sem0).start()

        def body(g, carry):
            for j in range(2):
                c = 2 * g + j
                buf, sem = bufs[j], sems[j]
                nbuf, nsem = bufs[1 - j], sems[1 - j]
                gather(c, buf, sem).wait()

                @pl.when(c + 1 < n_chunks)
                def _():
                    gather(c + 1, nbuf, nsem).start()

                pltpu.sync_copy(buf, out_hbm.at[pl.ds(base + c * CHUNK, CHUNK)])
            return carry

        lax.fori_loop(0, n_chunks // 2, body, 0)

    return k(flat_idx, embeddings)


def kernel(inputs, embeddings):
    shape = inputs.shape
    flat = jnp.reshape(inputs, (-1,)).astype(jnp.int32)
    out = _lookup(flat, embeddings, flat.shape[0])
    return jnp.reshape(out, tuple(shape) + (D,))


# speedup vs baseline: 3.3182x; 1.0624x over previous
import functools

import jax
import jax.numpy as jnp
from jax import lax
from jax.experimental import pallas as pl
from jax.experimental.pallas import tpu as pltpu
from jax.experimental.pallas import tpu_sc as plsc

VOCAB = 100000
D = 128
NUM_CORES = 2
NUM_SUBCORES = 16
NW = NUM_CORES * NUM_SUBCORES
CHUNK = 128
SUPER = 320
SUBS = ((0, 128), (128, 128), (256, 64))


@functools.partial(jax.jit, static_argnums=(2,))
def _lookup(flat_idx, embeddings, n):
    b_per_w = n // NW
    n_super = b_per_w // SUPER
    mesh = plsc.VectorSubcoreMesh(core_axis_name="c", subcore_axis_name="s")

    assert n_super % 2 == 0 and b_per_w % SUPER == 0

    @functools.partial(
        pl.kernel,
        mesh=mesh,
        out_type=jax.ShapeDtypeStruct((n, D), jnp.float32),
        scratch_types=[
            pltpu.VMEM((b_per_w,), jnp.int32),
            pltpu.VMEM((SUPER, D), jnp.float32),
            pltpu.VMEM((SUPER, D), jnp.float32),
            pltpu.SemaphoreType.DMA,
            pltpu.SemaphoreType.DMA,
        ],
    )
    def k(idx_hbm, table_hbm, out_hbm, idx_v, rows0, rows1, sem0, sem1):
        wid = lax.axis_index("s") * NUM_CORES + lax.axis_index("c")
        base = wid * b_per_w
        pltpu.sync_copy(idx_hbm.at[pl.ds(base, b_per_w)], idx_v)

        bufs = (rows0, rows1)
        sems = (sem0, sem1)

        def streams(s, buf, sem):
            return [
                pltpu.make_async_copy(
                    table_hbm.at[idx_v.at[pl.ds(s * SUPER + off, sz)]],
                    buf.at[pl.ds(off, sz)],
                    sem,
                )
                for off, sz in SUBS
            ]

        def fire(s, buf, sem):
            for st in streams(s, buf, sem):
                st.start()

        fire(0, rows0, sem0)

        def body(g, carry):
            for j in range(2):
                s = 2 * g + j
                buf, sem = bufs[j], sems[j]
                nbuf, nsem = bufs[1 - j], sems[1 - j]
                for st in streams(s, buf, sem):
                    st.wait()

                @pl.when(s + 1 < n_super)
                def _():
                    fire(s + 1, nbuf, nsem)

                pltpu.sync_copy(buf, out_hbm.at[pl.ds(base + s * SUPER, SUPER)])
            return carry

        lax.fori_loop(0, n_super // 2, body, 0)

    return k(flat_idx, embeddings)


def kernel(inputs, embeddings):
    shape = inputs.shape
    flat = jnp.reshape(inputs, (-1,)).astype(jnp.int32)
    out = _lookup(flat, embeddings, flat.shape[0])
    return jnp.reshape(out, tuple(shape) + (D,))
